# baseline (speedup 1.0000x reference)
import jax
import jax.numpy as jnp
from jax.experimental import pallas as pl
from jax.experimental.pallas import tpu as pltpu

_DT = 0.1
_HALF_DT = _DT / 2.0
_CG_MAX_ITER = 20
_CG_TOL = 1e-06
_DILS = (5, 10, 20)
_HALO = 24
_TILE = 512


def _store_haloed(sref, v, D):
    sref[pl.ds(_HALO, D), :] = v
    sref[pl.ds(0, _HALO), :] = v[D - _HALO:, :]
    sref[pl.ds(_HALO + D, _HALO), :] = v[:_HALO, :]


def _cayley_apply(sref, v_self, v_other, positive, hw, hpot, D):
    if positive:
        out = v_self + hpot * v_other
    else:
        out = v_self - hpot * v_other
    for s, d in enumerate(_DILS):
        pair = sref[pl.ds(_HALO - d, D), :] + sref[pl.ds(_HALO + d, D), :]
        if positive:
            out = out - hw[s] * pair
        else:
            out = out + hw[s] * pair
    return out


def _body(scale_ref, psir_ref, psii_ref, alpha_ref, pot_ref, outr_ref, outi_ref,
          sr_ref, si_ref):
    D = psir_ref.shape[0]
    pr = psir_ref[:]
    pi_ = psii_ref[:]
    inten = pr * pr + pi_ * pi_
    m = jnp.mean(inten, axis=0, keepdims=True)
    inten = inten / (m + 1e-08)
    phase = alpha_ref[:] * inten
    c = jnp.cos(phase)
    sn = jnp.sin(phase)
    rot_r = pr * c - pi_ * sn
    rot_i = pr * sn + pi_ * c

    hw = (_HALF_DT * scale_ref[0], _HALF_DT * scale_ref[1],
          _HALF_DT * scale_ref[2])
    hpot = _HALF_DT * pot_ref[:]

    _store_haloed(sr_ref, rot_r, D)
    _store_haloed(si_ref, rot_i, D)
    rhs_r = _cayley_apply(si_ref, rot_r, rot_i, True, hw, hpot, D)
    rhs_i = _cayley_apply(sr_ref, rot_i, rot_r, False, hw, hpot, D)

    def tokdot(ar, ai, br, bi):
        return jnp.sum(ar * br + ai * bi, axis=0, keepdims=True)

    r_r = rhs_r
    r_i = rhs_i
    outr_ref[:] = jnp.zeros_like(r_r)
    outi_ref[:] = jnp.zeros_like(r_i)
    rs_old = tokdot(r_r, r_i, r_r, r_i)
    _store_haloed(sr_ref, r_r, D)
    _store_haloed(si_ref, r_i, D)

    def cg_iter(it, carry):
        r_r, r_i, rs_old = carry
        active = jnp.sqrt(rs_old) > _CG_TOL
        p_r = sr_ref[pl.ds(_HALO, D), :]
        p_i = si_ref[pl.ds(_HALO, D), :]
        Ap_r = _cayley_apply(si_ref, p_r, p_i, False, hw, hpot, D)
        Ap_i = _cayley_apply(sr_ref, p_i, p_r, True, hw, hpot, D)
        pAp = tokdot(p_r, p_i, Ap_r, Ap_i)
        a = jnp.where(active, rs_old / (pAp + 1e-12), 0.0)
        outr_ref[:] += a * p_r
        outi_ref[:] += a * p_i
        r_r = r_r - a * Ap_r
        r_i = r_i - a * Ap_i
        rs_new = tokdot(r_r, r_i, r_r, r_i)
        beta = jnp.where(active, rs_new / (rs_old + 1e-12), 0.0)
        _store_haloed(sr_ref, r_r + beta * p_r, D)
        _store_haloed(si_ref, r_i + beta * p_i, D)
        rs_old = jnp.where(active, rs_new, rs_old)
        return (r_r, r_i, rs_old)

    carry = (r_r, r_i, rs_old)
    carry = jax.lax.fori_loop(0, _CG_MAX_ITER, cg_iter, carry, unroll=2)


def kernel(psi, alpha, scale_w, potential):
    b, s, d, _ = psi.shape
    n = b * s
    psir = psi[..., 0].reshape(n, d).T
    psii = psi[..., 1].reshape(n, d).T
    alpha2 = alpha.reshape(d, 1)
    pot_eff = (potential + 2.0 * jnp.sum(scale_w)).reshape(d, 1)

    grid = (n // _TILE,)
    out_r, out_i = pl.pallas_call(
        _body,
        grid=grid,
        in_specs=[
            pl.BlockSpec(memory_space=pltpu.SMEM),
            pl.BlockSpec((d, _TILE), lambda i: (0, i)),
            pl.BlockSpec((d, _TILE), lambda i: (0, i)),
            pl.BlockSpec((d, 1), lambda i: (0, 0)),
            pl.BlockSpec((d, 1), lambda i: (0, 0)),
        ],
        out_specs=[
            pl.BlockSpec((d, _TILE), lambda i: (0, i)),
            pl.BlockSpec((d, _TILE), lambda i: (0, i)),
        ],
        out_shape=[jax.ShapeDtypeStruct((d, n), jnp.float32)] * 2,
        scratch_shapes=[
            pltpu.VMEM((d + 2 * _HALO, _TILE), jnp.float32),
            pltpu.VMEM((d + 2 * _HALO, _TILE), jnp.float32),
        ],
        compiler_params=pltpu.CompilerParams(
            dimension_semantics=("arbitrary",),
        ),
    )(scale_w, psir, psii, alpha2, pot_eff)
    return jnp.stack([out_r.T, out_i.T], axis=-1).reshape(b, s, d, 2)
